# Initial kernel scaffold; baseline (speedup 1.0000x reference)
import jax
import jax.numpy as jnp
from jax.experimental import pallas as pl


def kernel(x, edge_index, W1, b1, W2, b2, proj_W, proj_b):
    raise NotImplementedError("write your pallas kernel here")


# speedup vs baseline: 10.5578x; 10.5578x over previous
import functools

import jax
import jax.numpy as jnp
from jax import lax
from jax.experimental import pallas as pl
from jax.experimental.pallas import tpu as pltpu
from jax.experimental.pallas import tpu_sc as plsc

N = 50000
E = 1600000
D = 128
H = 256
C = 40
K = 10

NPAD = 50688
ROWS_PER_TILE = NPAD // 16
ZROWS = 528
EPT = 50176
EPAD = 32 * EPT
E2D = EPAD // 128
GROUPS = EPT // 1024
TRASH = 63
PADC = 16383

_mesh = plsc.VectorSubcoreMesh(core_axis_name="c", subcore_axis_name="s")
_sc_params = pltpu.CompilerParams(use_tc_tiling_on_sc=False,
                                  internal_scratch_in_bytes=65536)


@functools.partial(
    pl.kernel,
    mesh=_mesh,
    compiler_params=_sc_params,
    out_type=[
        jax.ShapeDtypeStruct((EPAD,), jnp.int32),
        jax.ShapeDtypeStruct((2 * NPAD,), jnp.float32),
    ],
    scratch_types=[
        pltpu.VMEM((1024,), jnp.int32),
        pltpu.VMEM((8, 128), jnp.int32),
        pltpu.VMEM((1024,), jnp.int32),
        pltpu.VMEM((1024,), jnp.float32),
        pltpu.VMEM((ROWS_PER_TILE,), jnp.float32),
        pltpu.VMEM_SHARED((NPAD,), jnp.float32),
        pltpu.SemaphoreType.DMA,
    ],
)
def _sc_pre(rowp, colp2, z1, row2, degp, rbuf, cbuf2, r2buf,
            ewbuf, stage, degacc, sem):
    cid = lax.axis_index("c")
    sid = lax.axis_index("s")
    wid = sid * 2 + cid
    s0 = sid * ROWS_PER_TILE

    pltpu.sync_copy(z1.at[pl.ds(s0, ROWS_PER_TILE)], stage)
    pltpu.sync_copy(stage, degacc.at[pl.ds(s0, ROWS_PER_TILE)])
    plsc.subcore_barrier()

    lanes = lax.iota(jnp.int32, 16)

    def _group(g, _):
        e_base = pl.multiple_of(wid * EPT + g * 1024, 1024)
        pltpu.sync_copy(rowp.at[pl.ds(e_base, 1024)], rbuf)
        pltpu.sync_copy(colp2.at[pl.ds(pl.multiple_of(e_base // 128, 8), 8)],
                        cbuf2)

        def _vec(t, _):
            r = rbuf[pl.ds(t * 16, 16)]
            c = cbuf2[t // 8, pl.ds((t % 8) * 16, 16)]
            e = e_base + t * 16 + lanes
            selfm = r == c
            r2buf[pl.ds(t * 16, 16)] = jnp.where(selfm, N + (e & TRASH), r)
            ewbuf[pl.ds(t * 16, 16)] = jnp.where(selfm, 0.0, 1.0)
            return _
        lax.fori_loop(0, 64, _vec, 0)

        pltpu.sync_copy(r2buf, row2.at[pl.ds(e_base, 1024)])
        for j in range(8):
            pltpu.sync_copy(ewbuf.at[pl.ds(j * 128, 128)],
                            degacc.at[cbuf2.at[j]], add=True)
        return _
    lax.fori_loop(0, GROUPS, _group, 0)

    plsc.subcore_barrier()
    pltpu.sync_copy(degacc.at[pl.ds(s0, ROWS_PER_TILE)], stage)
    pltpu.sync_copy(
        stage,
        degp.at[pl.ds(pl.multiple_of(cid * NPAD + s0, 8), ROWS_PER_TILE)])


CH = C // 2


@functools.partial(
    pl.kernel,
    mesh=_mesh,
    compiler_params=_sc_params,
    out_type=jax.ShapeDtypeStruct((2, 2, NPAD, CH), jnp.float32),
    scratch_types=[
        pltpu.VMEM((1024,), jnp.int32),
        pltpu.VMEM((8, 128), jnp.int32),
        pltpu.VMEM((128, CH), jnp.float32),
        pltpu.VMEM((ZROWS, CH), jnp.float32),
        pltpu.VMEM_SHARED((NPAD, CH), jnp.float32),
        pltpu.SemaphoreType.DMA,
    ],
)
def _sc_prop(curtA, curtB, row2, colp, zc, pout, ridx, cidx, rows, zrow, acc,
             sem):
    cid = lax.axis_index("c")
    sid = lax.axis_index("s")
    wid = sid * 2 + cid
    s0 = sid * ROWS_PER_TILE

    for h, curt in ((0, curtA), (1, curtB)):
        pltpu.sync_copy(zc, zrow)
        for t in range(ROWS_PER_TILE // ZROWS):
            pltpu.sync_copy(zrow, acc.at[pl.ds(s0 + t * ZROWS, ZROWS)])
        plsc.subcore_barrier()

        def _group(g, _):
            e_base = pl.multiple_of(wid * EPT + g * 1024, 1024)
            pltpu.sync_copy(row2.at[pl.ds(e_base, 1024)], ridx)
            pltpu.sync_copy(
                colp.at[pl.ds(pl.multiple_of(e_base // 128, 8), 8)], cidx)
            for j in range(8):
                pltpu.async_copy(
                    curt.at[ridx.at[pl.ds(j * 128, 128)]], rows, sem).wait()
                pltpu.sync_copy(rows, acc.at[cidx.at[j]], add=True)
            return _
        lax.fori_loop(0, GROUPS, _group, 0)

        plsc.subcore_barrier()
        for t in range(ROWS_PER_TILE // ZROWS):
            r0 = s0 + t * ZROWS
            pltpu.sync_copy(acc.at[pl.ds(r0, ZROWS)], zrow)
            pltpu.sync_copy(zrow, pout.at[h, cid, pl.ds(r0, ZROWS)])
        plsc.subcore_barrier()


_NB = 352
_GRID = NPAD // _NB


def _mlp_body(x_ref, w1_ref, b1_ref, w2_ref, b2_ref, degp_ref, pw_ref,
              pb_ref, cur_ref, curta_ref, curtb_ref, oacc_ref):
    h1 = jnp.maximum(
        jnp.dot(x_ref[...], w1_ref[...], preferred_element_type=jnp.float32)
        + b1_ref[...], 0.0)
    h = (jnp.dot(h1, w2_ref[...], preferred_element_type=jnp.float32)
         + b2_ref[...])
    rows = (pl.program_id(0) * _NB
            + lax.broadcasted_iota(jnp.int32, (_NB, 1), 0))
    h = jnp.where(rows < N, h, 0.0)
    deg = degp_ref[0] + degp_ref[1] + 1.0
    dinv = lax.rsqrt(deg)
    cur_ref[...] = h
    ct = h * dinv
    curta_ref[...] = ct[:, :CH]
    curtb_ref[...] = ct[:, CH:]
    r = jax.nn.sigmoid(
        jnp.dot(h, pw_ref[...], preferred_element_type=jnp.float32)
        + pb_ref[...])
    oacc_ref[...] = r * h


_mlp = pl.pallas_call(
    _mlp_body,
    grid=(_GRID,),
    in_specs=[
        pl.BlockSpec((_NB, D), lambda i: (i, 0)),
        pl.BlockSpec((D, H), lambda i: (0, 0)),
        pl.BlockSpec((1, H), lambda i: (0, 0)),
        pl.BlockSpec((H, C), lambda i: (0, 0)),
        pl.BlockSpec((1, C), lambda i: (0, 0)),
        pl.BlockSpec((2, _NB, 1), lambda i: (0, i, 0)),
        pl.BlockSpec((C, 1), lambda i: (0, 0)),
        pl.BlockSpec((1, 1), lambda i: (0, 0)),
    ],
    out_specs=[
        pl.BlockSpec((_NB, C), lambda i: (i, 0)),
        pl.BlockSpec((_NB, CH), lambda i: (i, 0)),
        pl.BlockSpec((_NB, CH), lambda i: (i, 0)),
        pl.BlockSpec((_NB, C), lambda i: (i, 0)),
    ],
    out_shape=[
        jax.ShapeDtypeStruct((NPAD, C), jnp.float32),
        jax.ShapeDtypeStruct((NPAD, CH), jnp.float32),
        jax.ShapeDtypeStruct((NPAD, CH), jnp.float32),
        jax.ShapeDtypeStruct((NPAD, C), jnp.float32),
    ],
)


def _step_body(cur_ref, p_ref, degp_ref, pw_ref, pb_ref, oin_ref,
               cur_ref_o, curta_ref_o, curtb_ref_o, oacc_ref_o):
    deg = degp_ref[0] + degp_ref[1] + 1.0
    dinv = lax.rsqrt(deg)
    ideg = 1.0 / deg
    s = jnp.concatenate(
        [p_ref[0, 0] + p_ref[0, 1], p_ref[1, 0] + p_ref[1, 1]], axis=1)
    c = dinv * s + ideg * cur_ref[...]
    cur_ref_o[...] = c
    ct = c * dinv
    curta_ref_o[...] = ct[:, :CH]
    curtb_ref_o[...] = ct[:, CH:]
    r = jax.nn.sigmoid(
        jnp.dot(c, pw_ref[...], preferred_element_type=jnp.float32)
        + pb_ref[...])
    oacc_ref_o[...] = oin_ref[...] + r * c


_step = pl.pallas_call(
    _step_body,
    grid=(_GRID,),
    in_specs=[
        pl.BlockSpec((_NB, C), lambda i: (i, 0)),
        pl.BlockSpec((2, 2, _NB, CH), lambda i: (0, 0, i, 0)),
        pl.BlockSpec((2, _NB, 1), lambda i: (0, i, 0)),
        pl.BlockSpec((C, 1), lambda i: (0, 0)),
        pl.BlockSpec((1, 1), lambda i: (0, 0)),
        pl.BlockSpec((_NB, C), lambda i: (i, 0)),
    ],
    out_specs=[
        pl.BlockSpec((_NB, C), lambda i: (i, 0)),
        pl.BlockSpec((_NB, CH), lambda i: (i, 0)),
        pl.BlockSpec((_NB, CH), lambda i: (i, 0)),
        pl.BlockSpec((_NB, C), lambda i: (i, 0)),
    ],
    out_shape=[
        jax.ShapeDtypeStruct((NPAD, C), jnp.float32),
        jax.ShapeDtypeStruct((NPAD, CH), jnp.float32),
        jax.ShapeDtypeStruct((NPAD, CH), jnp.float32),
        jax.ShapeDtypeStruct((NPAD, C), jnp.float32),
    ],
)


def _final_body(o_ref, out_ref):
    o = o_ref[...]
    m = jnp.max(o, axis=1, keepdims=True)
    e = jnp.exp(o - m)
    out_ref[...] = o - m - jnp.log(jnp.sum(e, axis=1, keepdims=True))


_final = pl.pallas_call(
    _final_body,
    grid=(N // 400,),
    in_specs=[pl.BlockSpec((400, C), lambda i: (i, 0))],
    out_specs=pl.BlockSpec((400, C), lambda i: (i, 0)),
    out_shape=jax.ShapeDtypeStruct((N, C), jnp.float32),
)


def kernel(x, edge_index, W1, b1, W2, b2, proj_W, proj_b):
    row = edge_index[0]
    col = edge_index[1]
    pad = (jnp.arange(E, EPAD, dtype=jnp.int32) & PADC)
    rowp = jnp.concatenate([row, pad])
    colp2 = jnp.concatenate([col, pad]).reshape(E2D, 128)
    z1 = jnp.zeros((NPAD,), jnp.float32)
    zc = jnp.zeros((ZROWS, CH), jnp.float32)
    x_p = jnp.zeros((NPAD, D), jnp.float32).at[:N].set(x)
    b1r = b1.reshape(1, H)
    b2r = b2.reshape(1, C)
    pbr = proj_b.reshape(1, 1)

    row2, degp = _sc_pre(rowp, colp2, z1)
    degp3 = degp.reshape(2, NPAD, 1)
    cur, curta, curtb, oacc = _mlp(x_p, W1, b1r, W2, b2r, degp3, proj_W, pbr)
    for _ in range(K):
        p = _sc_prop(curta, curtb, row2, colp2, zc)
        cur, curta, curtb, oacc = _step(cur, p, degp3, proj_W, pbr, oacc)
    return _final(oacc)


# speedup vs baseline: 11.6410x; 1.1026x over previous
import functools

import jax
import jax.numpy as jnp
from jax import lax
from jax.experimental import pallas as pl
from jax.experimental.pallas import tpu as pltpu
from jax.experimental.pallas import tpu_sc as plsc

N = 50000
E = 1600000
D = 128
H = 256
C = 40
K = 10

NPAD = 50688
ROWS_PER_TILE = NPAD // 16
ZROWS = 528
EPT = 50176
EPAD = 32 * EPT
E2D = EPAD // 128
GROUPS = EPT // 1024
TRASH = 63
PADC = 16383

_mesh = plsc.VectorSubcoreMesh(core_axis_name="c", subcore_axis_name="s")
_sc_params = pltpu.CompilerParams(use_tc_tiling_on_sc=False,
                                  internal_scratch_in_bytes=65536)


@functools.partial(
    pl.kernel,
    mesh=_mesh,
    compiler_params=_sc_params,
    out_type=[
        jax.ShapeDtypeStruct((EPAD,), jnp.int32),
        jax.ShapeDtypeStruct((2 * NPAD,), jnp.float32),
    ],
    scratch_types=[
        pltpu.VMEM((1024,), jnp.int32),
        pltpu.VMEM((8, 128), jnp.int32),
        pltpu.VMEM((1024,), jnp.int32),
        pltpu.VMEM((1024,), jnp.float32),
        pltpu.VMEM((ROWS_PER_TILE,), jnp.float32),
        pltpu.VMEM_SHARED((NPAD,), jnp.float32),
        pltpu.SemaphoreType.DMA,
    ],
)
def _sc_pre(rowp, colp2, z1, row2, degp, rbuf, cbuf2, r2buf,
            ewbuf, stage, degacc, sem):
    cid = lax.axis_index("c")
    sid = lax.axis_index("s")
    wid = sid * 2 + cid
    s0 = sid * ROWS_PER_TILE

    pltpu.sync_copy(z1.at[pl.ds(s0, ROWS_PER_TILE)], stage)
    pltpu.sync_copy(stage, degacc.at[pl.ds(s0, ROWS_PER_TILE)])
    plsc.subcore_barrier()

    lanes = lax.iota(jnp.int32, 16)

    def _group(g, _):
        e_base = pl.multiple_of(wid * EPT + g * 1024, 1024)
        pltpu.sync_copy(rowp.at[pl.ds(e_base, 1024)], rbuf)
        pltpu.sync_copy(colp2.at[pl.ds(pl.multiple_of(e_base // 128, 8), 8)],
                        cbuf2)

        def _vec(t, _):
            r = rbuf[pl.ds(t * 16, 16)]
            c = cbuf2[t // 8, pl.ds((t % 8) * 16, 16)]
            e = e_base + t * 16 + lanes
            selfm = r == c
            r2buf[pl.ds(t * 16, 16)] = jnp.where(selfm, N + (e & TRASH), r)
            ewbuf[pl.ds(t * 16, 16)] = jnp.where(selfm, 0.0, 1.0)
            return _
        lax.fori_loop(0, 64, _vec, 0)

        pltpu.sync_copy(r2buf, row2.at[pl.ds(e_base, 1024)])
        for j in range(8):
            pltpu.sync_copy(ewbuf.at[pl.ds(j * 128, 128)],
                            degacc.at[cbuf2.at[j]], add=True)
        return _
    lax.fori_loop(0, GROUPS, _group, 0)

    plsc.subcore_barrier()
    pltpu.sync_copy(degacc.at[pl.ds(s0, ROWS_PER_TILE)], stage)
    pltpu.sync_copy(
        stage,
        degp.at[pl.ds(pl.multiple_of(cid * NPAD + s0, 8), ROWS_PER_TILE)])


CH = C // 2


@functools.partial(
    pl.kernel,
    mesh=_mesh,
    compiler_params=_sc_params,
    out_type=jax.ShapeDtypeStruct((2, 2, NPAD, CH), jnp.float32),
    scratch_types=[
        pltpu.VMEM((1024,), jnp.int32),
        pltpu.VMEM((8, 128), jnp.int32),
        pltpu.VMEM((128, CH), jnp.float32),
        pltpu.VMEM((128, CH), jnp.float32),
        pltpu.VMEM((ZROWS, CH), jnp.float32),
        pltpu.VMEM_SHARED((NPAD, CH), jnp.float32),
        pltpu.SemaphoreType.DMA,
        pltpu.SemaphoreType.DMA,
    ],
)
def _sc_prop(curtA, curtB, row2, colp, zc, pout, ridx, cidx, rows0, rows1,
             zrow, acc, sem0, sem1):
    cid = lax.axis_index("c")
    sid = lax.axis_index("s")
    wid = sid * 2 + cid
    s0 = sid * ROWS_PER_TILE

    for h, curt in ((0, curtA), (1, curtB)):
        pltpu.sync_copy(zc, zrow)
        for t in range(ROWS_PER_TILE // ZROWS):
            pltpu.sync_copy(zrow, acc.at[pl.ds(s0 + t * ZROWS, ZROWS)])
        plsc.subcore_barrier()

        bufs = (rows0, rows1)
        sems = (sem0, sem1)

        def _group(g, _):
            e_base = pl.multiple_of(wid * EPT + g * 1024, 1024)
            pltpu.sync_copy(row2.at[pl.ds(e_base, 1024)], ridx)
            pltpu.sync_copy(
                colp.at[pl.ds(pl.multiple_of(e_base // 128, 8), 8)], cidx)
            h = pltpu.async_copy(
                curt.at[ridx.at[pl.ds(0, 128)]], bufs[0], sems[0])
            for j in range(8):
                h.wait()
                if j < 7:
                    h = pltpu.async_copy(
                        curt.at[ridx.at[pl.ds((j + 1) * 128, 128)]],
                        bufs[(j + 1) % 2], sems[(j + 1) % 2])
                pltpu.sync_copy(bufs[j % 2], acc.at[cidx.at[j]], add=True)
            return _
        lax.fori_loop(0, GROUPS, _group, 0)

        plsc.subcore_barrier()
        for t in range(ROWS_PER_TILE // ZROWS):
            r0 = s0 + t * ZROWS
            pltpu.sync_copy(acc.at[pl.ds(r0, ZROWS)], zrow)
            pltpu.sync_copy(zrow, pout.at[h, cid, pl.ds(r0, ZROWS)])
        plsc.subcore_barrier()


_NB = 352
_GRID = NPAD // _NB


def _mlp_body(x_ref, w1_ref, b1_ref, w2_ref, b2_ref, degp_ref, pw_ref,
              pb_ref, cur_ref, curta_ref, curtb_ref, oacc_ref):
    h1 = jnp.maximum(
        jnp.dot(x_ref[...], w1_ref[...], preferred_element_type=jnp.float32)
        + b1_ref[...], 0.0)
    h = (jnp.dot(h1, w2_ref[...], preferred_element_type=jnp.float32)
         + b2_ref[...])
    rows = (pl.program_id(0) * _NB
            + lax.broadcasted_iota(jnp.int32, (_NB, 1), 0))
    h = jnp.where(rows < N, h, 0.0)
    deg = degp_ref[0] + degp_ref[1] + 1.0
    dinv = lax.rsqrt(deg)
    cur_ref[...] = h
    ct = h * dinv
    curta_ref[...] = ct[:, :CH]
    curtb_ref[...] = ct[:, CH:]
    r = jax.nn.sigmoid(
        jnp.dot(h, pw_ref[...], preferred_element_type=jnp.float32)
        + pb_ref[...])
    oacc_ref[...] = r * h


_mlp = pl.pallas_call(
    _mlp_body,
    grid=(_GRID,),
    in_specs=[
        pl.BlockSpec((_NB, D), lambda i: (i, 0)),
        pl.BlockSpec((D, H), lambda i: (0, 0)),
        pl.BlockSpec((1, H), lambda i: (0, 0)),
        pl.BlockSpec((H, C), lambda i: (0, 0)),
        pl.BlockSpec((1, C), lambda i: (0, 0)),
        pl.BlockSpec((2, _NB, 1), lambda i: (0, i, 0)),
        pl.BlockSpec((C, 1), lambda i: (0, 0)),
        pl.BlockSpec((1, 1), lambda i: (0, 0)),
    ],
    out_specs=[
        pl.BlockSpec((_NB, C), lambda i: (i, 0)),
        pl.BlockSpec((_NB, CH), lambda i: (i, 0)),
        pl.BlockSpec((_NB, CH), lambda i: (i, 0)),
        pl.BlockSpec((_NB, C), lambda i: (i, 0)),
    ],
    out_shape=[
        jax.ShapeDtypeStruct((NPAD, C), jnp.float32),
        jax.ShapeDtypeStruct((NPAD, CH), jnp.float32),
        jax.ShapeDtypeStruct((NPAD, CH), jnp.float32),
        jax.ShapeDtypeStruct((NPAD, C), jnp.float32),
    ],
)


def _step_body(cur_ref, p_ref, degp_ref, pw_ref, pb_ref, oin_ref,
               cur_ref_o, curta_ref_o, curtb_ref_o, oacc_ref_o):
    deg = degp_ref[0] + degp_ref[1] + 1.0
    dinv = lax.rsqrt(deg)
    ideg = 1.0 / deg
    s = jnp.concatenate(
        [p_ref[0, 0] + p_ref[0, 1], p_ref[1, 0] + p_ref[1, 1]], axis=1)
    c = dinv * s + ideg * cur_ref[...]
    cur_ref_o[...] = c
    ct = c * dinv
    curta_ref_o[...] = ct[:, :CH]
    curtb_ref_o[...] = ct[:, CH:]
    r = jax.nn.sigmoid(
        jnp.dot(c, pw_ref[...], preferred_element_type=jnp.float32)
        + pb_ref[...])
    oacc_ref_o[...] = oin_ref[...] + r * c


_step = pl.pallas_call(
    _step_body,
    grid=(_GRID,),
    in_specs=[
        pl.BlockSpec((_NB, C), lambda i: (i, 0)),
        pl.BlockSpec((2, 2, _NB, CH), lambda i: (0, 0, i, 0)),
        pl.BlockSpec((2, _NB, 1), lambda i: (0, i, 0)),
        pl.BlockSpec((C, 1), lambda i: (0, 0)),
        pl.BlockSpec((1, 1), lambda i: (0, 0)),
        pl.BlockSpec((_NB, C), lambda i: (i, 0)),
    ],
    out_specs=[
        pl.BlockSpec((_NB, C), lambda i: (i, 0)),
        pl.BlockSpec((_NB, CH), lambda i: (i, 0)),
        pl.BlockSpec((_NB, CH), lambda i: (i, 0)),
        pl.BlockSpec((_NB, C), lambda i: (i, 0)),
    ],
    out_shape=[
        jax.ShapeDtypeStruct((NPAD, C), jnp.float32),
        jax.ShapeDtypeStruct((NPAD, CH), jnp.float32),
        jax.ShapeDtypeStruct((NPAD, CH), jnp.float32),
        jax.ShapeDtypeStruct((NPAD, C), jnp.float32),
    ],
)


def _final_body(o_ref, out_ref):
    o = o_ref[...]
    m = jnp.max(o, axis=1, keepdims=True)
    e = jnp.exp(o - m)
    out_ref[...] = o - m - jnp.log(jnp.sum(e, axis=1, keepdims=True))


_final = pl.pallas_call(
    _final_body,
    grid=(N // 400,),
    in_specs=[pl.BlockSpec((400, C), lambda i: (i, 0))],
    out_specs=pl.BlockSpec((400, C), lambda i: (i, 0)),
    out_shape=jax.ShapeDtypeStruct((N, C), jnp.float32),
)


def kernel(x, edge_index, W1, b1, W2, b2, proj_W, proj_b):
    row = edge_index[0]
    col = edge_index[1]
    pad = (jnp.arange(E, EPAD, dtype=jnp.int32) & PADC)
    rowp = jnp.concatenate([row, pad])
    colp2 = jnp.concatenate([col, pad]).reshape(E2D, 128)
    z1 = jnp.zeros((NPAD,), jnp.float32)
    zc = jnp.zeros((ZROWS, CH), jnp.float32)
    x_p = jnp.zeros((NPAD, D), jnp.float32).at[:N].set(x)
    b1r = b1.reshape(1, H)
    b2r = b2.reshape(1, C)
    pbr = proj_b.reshape(1, 1)

    row2, degp = _sc_pre(rowp, colp2, z1)
    degp3 = degp.reshape(2, NPAD, 1)
    cur, curta, curtb, oacc = _mlp(x_p, W1, b1r, W2, b2r, degp3, proj_W, pbr)
    for _ in range(K):
        p = _sc_prop(curta, curtb, row2, colp2, zc)
        cur, curta, curtb, oacc = _step(cur, p, degp3, proj_W, pbr, oacc)
    return _final(oacc)
